# Initial kernel scaffold; baseline (speedup 1.0000x reference)
import jax
import jax.numpy as jnp
from jax.experimental import pallas as pl


def kernel(node_feats, edge_feats, edge_index, W1, b1, We1, be1, We2, be2, Wn1, bn1, Wn2, bn2):
    raise NotImplementedError("write your pallas kernel here")


# speedup vs baseline: 1.3601x; 1.3601x over previous
import functools

import jax
import jax.numpy as jnp
from jax import lax
from jax.experimental import pallas as pl
from jax.experimental.pallas import tpu as pltpu
from jax.experimental.pallas import tpu_sc as plsc

_LOG2 = 0.6931471805599453

_NUM_CORES = 2
_NUM_SUBCORES = 16

_IDXW = 128
_CROWS = 2


def _ssp(x):
    return jnp.maximum(x, 0.0) + jnp.log1p(jnp.exp(-jnp.abs(x))) - _LOG2


def _mlp_block_kernel(x_ref, wa_ref, ba_ref, wb_ref, bb_ref, o_ref):
    z = jnp.dot(x_ref[...], wa_ref[...], preferred_element_type=jnp.float32)
    z = _ssp(z + ba_ref[...])
    o_ref[...] = (
        jnp.dot(z, wb_ref[...], preferred_element_type=jnp.float32) + bb_ref[...]
    )


def _mlp(x, wa_t, ba, wb_t, bb, block_rows):
    rows, dim = x.shape
    assert rows % block_rows == 0
    return pl.pallas_call(
        _mlp_block_kernel,
        grid=(rows // block_rows,),
        in_specs=[
            pl.BlockSpec((block_rows, dim), lambda i: (i, 0)),
            pl.BlockSpec(wa_t.shape, lambda i: (0, 0)),
            pl.BlockSpec((1, dim), lambda i: (0, 0)),
            pl.BlockSpec(wb_t.shape, lambda i: (0, 0)),
            pl.BlockSpec((1, dim), lambda i: (0, 0)),
        ],
        out_specs=pl.BlockSpec((block_rows, dim), lambda i: (i, 0)),
        out_shape=jax.ShapeDtypeStruct((rows, dim), jnp.float32),
    )(x, wa_t, ba.reshape(1, dim), wb_t, bb.reshape(1, dim))


def _seg_sum(he, dst2d, zeros, n, dim):
    e_pad = he.shape[0]
    half = n // _NUM_CORES
    half_pad = half + 88
    n_rows = e_pad // _IDXW
    rows_per_sub = n_rows // _NUM_SUBCORES
    wb_chunk = 200
    n_wb = half // wb_chunk
    mesh = plsc.VectorSubcoreMesh(core_axis_name="c", subcore_axis_name="s")

    @functools.partial(
        pl.kernel,
        mesh=mesh,
        compiler_params=pltpu.CompilerParams(use_tc_tiling_on_sc=False),
        out_type=jax.ShapeDtypeStruct((n, dim), jnp.float32),
        scratch_types=[
            pltpu.VMEM((_CROWS, _IDXW), jnp.int32),
            pltpu.VMEM((_CROWS, _IDXW), jnp.int32),
            pltpu.VMEM((_CROWS * _IDXW, dim), jnp.float32),
            pltpu.VMEM_SHARED((half_pad, dim), jnp.float32),
        ],
    )
    def seg_kernel(he_hbm, dst_hbm, z_hbm, h_hbm, idx_raw, idx_loc, rows, acc):
        c = lax.axis_index("c")
        s = lax.axis_index("s")
        base_node = c * half

        init_rows = half_pad // _NUM_SUBCORES
        pltpu.sync_copy(
            z_hbm.at[pl.ds(s * init_rows, init_rows)],
            acc.at[pl.ds(s * init_rows, init_rows)],
        )
        plsc.subcore_barrier()

        @pl.loop(0, rows_per_sub, step=_CROWS)
        def _(i):
            row0 = s * rows_per_sub + i
            pltpu.sync_copy(dst_hbm.at[pl.ds(row0, _CROWS)], idx_raw)
            pltpu.sync_copy(he_hbm.at[pl.ds(row0 * _IDXW, _CROWS * _IDXW)], rows)
            for r in range(_CROWS):
                for g in range(_IDXW // 16):
                    v = idx_raw[r, pl.ds(g * 16, 16)]
                    lo = v - base_node
                    ok = (lo >= 0) & (lo < half)
                    idx_loc[r, pl.ds(g * 16, 16)] = jnp.where(ok, lo, half)
            for r in range(_CROWS):
                pltpu.sync_copy(
                    rows.at[pl.ds(r * _IDXW, _IDXW)],
                    acc.at[idx_loc.at[r]],
                    add=True,
                )

        plsc.subcore_barrier()

        @pl.loop(s, n_wb, step=_NUM_SUBCORES)
        def _(k):
            pltpu.sync_copy(
                acc.at[pl.ds(k * wb_chunk, wb_chunk)],
                h_hbm.at[pl.ds(base_node + k * wb_chunk, wb_chunk)],
            )

    return seg_kernel(he, dst2d, zeros)


def kernel(node_feats, edge_feats, edge_index, W1, b1, We1, be1, We2, be2,
           Wn1, bn1, Wn2, bn2):
    n, dim = node_feats.shape
    e = edge_feats.shape[0]

    e_pad = ((e + 16383) // 16384) * 16384
    pad = e_pad - e
    ef_pad = jnp.concatenate(
        [edge_feats, jnp.zeros((pad, edge_feats.shape[1]), jnp.float32)]
    )
    dst = edge_index[1]
    dst_pad = jnp.concatenate([dst, jnp.full((pad,), n, jnp.int32)])
    dst2d = dst_pad.reshape(e_pad // _IDXW, _IDXW)

    he = _mlp(ef_pad, We1.T, be1, We2.T, be2, block_rows=2048)

    half_pad = n // _NUM_CORES + 88
    zeros = jnp.zeros((half_pad, dim), jnp.float32)
    h = _seg_sum(he, dst2d, zeros, n, dim)

    return _mlp(h, Wn1.T, bn1, Wn2.T, bn2, block_rows=1000)


# speedup vs baseline: 1.3887x; 1.0210x over previous
import functools

import jax
import jax.numpy as jnp
from jax import lax
from jax.experimental import pallas as pl
from jax.experimental.pallas import tpu as pltpu
from jax.experimental.pallas import tpu_sc as plsc

_LOG2 = 0.6931471805599453

_NUM_CORES = 2
_NUM_SUBCORES = 16

_IDXW = 128
_IROWS = 14


def _ssp(x):
    return jnp.maximum(x, 0.0) + jnp.log1p(jnp.exp(-jnp.abs(x))) - _LOG2


def _mlp_block_kernel(x_ref, wa_ref, ba_ref, wb_ref, bb_ref, o_ref):
    z = jnp.dot(x_ref[...], wa_ref[...], preferred_element_type=jnp.float32)
    z = _ssp(z + ba_ref[...])
    o_ref[...] = (
        jnp.dot(z, wb_ref[...], preferred_element_type=jnp.float32) + bb_ref[...]
    )


def _mlp(x, wa_t, ba, wb_t, bb, block_rows):
    rows, dim = x.shape
    assert rows % block_rows == 0
    return pl.pallas_call(
        _mlp_block_kernel,
        grid=(rows // block_rows,),
        in_specs=[
            pl.BlockSpec((block_rows, dim), lambda i: (i, 0)),
            pl.BlockSpec(wa_t.shape, lambda i: (0, 0)),
            pl.BlockSpec((1, dim), lambda i: (0, 0)),
            pl.BlockSpec(wb_t.shape, lambda i: (0, 0)),
            pl.BlockSpec((1, dim), lambda i: (0, 0)),
        ],
        out_specs=pl.BlockSpec((block_rows, dim), lambda i: (i, 0)),
        out_shape=jax.ShapeDtypeStruct((rows, dim), jnp.float32),
    )(x, wa_t, ba.reshape(1, dim), wb_t, bb.reshape(1, dim))


def _seg_sum(he, dst2d, n, dim):
    e_pad = he.shape[0]
    half = n // _NUM_CORES
    half_pad = half + 88
    n_rows = e_pad // _IDXW
    rows_per_sub = n_rows // _NUM_SUBCORES
    wb_chunk = 200
    n_wb = half // wb_chunk
    mesh = plsc.VectorSubcoreMesh(core_axis_name="c", subcore_axis_name="s")

    @functools.partial(
        pl.kernel,
        mesh=mesh,
        compiler_params=pltpu.CompilerParams(use_tc_tiling_on_sc=False),
        out_type=jax.ShapeDtypeStruct((n, dim), jnp.float32),
        scratch_types=[
            pltpu.VMEM((_IROWS, _IDXW), jnp.int32),
            pltpu.VMEM((_IROWS, _IDXW), jnp.int32),
            pltpu.VMEM((_IDXW, dim), jnp.float32),
            pltpu.VMEM((_IDXW, dim), jnp.float32),
            pltpu.VMEM_SHARED((half_pad, dim), jnp.float32),
            pltpu.SemaphoreType.DMA,
            pltpu.SemaphoreType.DMA,
        ],
    )
    def seg_kernel(he_hbm, dst_hbm, h_hbm, idx_raw, idx_loc, buf0, buf1, acc,
                   sem0, sem1):
        c = lax.axis_index("c")
        s = lax.axis_index("s")
        base_node = c * half
        bufs = (buf0, buf1)
        sems = (sem0, sem1)
        chunk0 = s * rows_per_sub

        @pl.loop(0, _IDXW)
        def _(r):
            for g in range(dim // 16):
                buf0[r, pl.ds(g * 16, 16)] = jnp.zeros((16,), jnp.float32)

        init_rows = half_pad // _NUM_SUBCORES
        @pl.loop(0, 14)
        def _(k):
            pltpu.sync_copy(
                buf0.at[pl.ds(0, 112)],
                acc.at[pl.ds(s * init_rows + k * 112, 112)],
            )
        plsc.subcore_barrier()

        pltpu.async_copy(he_hbm.at[pl.ds(chunk0 * _IDXW, _IDXW)], buf0, sem0)
        pltpu.async_copy(he_hbm.at[pl.ds((chunk0 + 1) * _IDXW, _IDXW)], buf1, sem1)

        @pl.loop(0, rows_per_sub, step=_IROWS)
        def _(i):
            pltpu.sync_copy(dst_hbm.at[pl.ds(chunk0 + i, _IROWS)], idx_raw)
            for r in range(_IROWS):
                for g in range(_IDXW // 16):
                    v = idx_raw[r, pl.ds(g * 16, 16)]
                    lo = v - base_node
                    ok = (lo >= 0) & (lo < half)
                    idx_loc[r, pl.ds(g * 16, 16)] = jnp.where(ok, lo, half)
            for k in range(_IROWS):
                b = k % 2
                src = he_hbm.at[pl.ds((chunk0 + i + k) * _IDXW, _IDXW)]
                pltpu.make_async_copy(src, bufs[b], sems[b]).wait()
                pltpu.sync_copy(bufs[b], acc.at[idx_loc.at[k]], add=True)

                @pl.when(i + k + 2 < rows_per_sub)
                def _():
                    pltpu.async_copy(
                        he_hbm.at[pl.ds((chunk0 + i + k + 2) * _IDXW, _IDXW)],
                        bufs[b],
                        sems[b],
                    )

        plsc.subcore_barrier()

        @pl.loop(s, n_wb, step=_NUM_SUBCORES)
        def _(k):
            pltpu.sync_copy(
                acc.at[pl.ds(k * wb_chunk, wb_chunk)],
                h_hbm.at[pl.ds(base_node + k * wb_chunk, wb_chunk)],
            )

    return seg_kernel(he, dst2d)


def kernel(node_feats, edge_feats, edge_index, W1, b1, We1, be1, We2, be2,
           Wn1, bn1, Wn2, bn2):
    n, dim = node_feats.shape
    e = edge_feats.shape[0]

    e_pad = ((e + 16383) // 16384) * 16384
    pad = e_pad - e
    ef_pad = jnp.concatenate(
        [edge_feats, jnp.zeros((pad, edge_feats.shape[1]), jnp.float32)]
    )
    dst = edge_index[1]
    dst_pad = jnp.concatenate([dst, jnp.full((pad,), n, jnp.int32)])
    dst2d = dst_pad.reshape(e_pad // _IDXW, _IDXW)

    he = _mlp(ef_pad, We1.T, be1, We2.T, be2, block_rows=2048)

    h = _seg_sum(he, dst2d, n, dim)

    return _mlp(h, Wn1.T, bn1, Wn2.T, bn2, block_rows=1000)
